# Initial kernel scaffold; baseline (speedup 1.0000x reference)
import jax
import jax.numpy as jnp
from jax.experimental import pallas as pl


def kernel(x, edge_index, edge_type, batch, W1, root1, b1, W2, root2, b2, gate_W, gate_b, fc_W, fc_b):
    raise NotImplementedError("write your pallas kernel here")


# speedup vs baseline: 13.5890x; 13.5890x over previous
import functools
import jax
import jax.numpy as jnp
from jax import lax
from jax.experimental import pallas as pl
from jax.experimental.pallas import tpu as pltpu
from jax.experimental.pallas import tpu_sc as plsc

N = 10000
E = 320000
R = 9
DIN = 5
D = 128
G = 64

NC = 2
NS = 16
NW = NC * NS
CH = 128
KCH = 80
EPT = CH * KCH
EPAD = EPT * NW
NR = N * R
ROWS_A = 5632
NRP = ROWS_A * NS
ROWS_C = 632
NP = ROWS_C * NS

_mesh = plsc.VectorSubcoreMesh(core_axis_name="c", subcore_axis_name="s",
                               num_cores=NC, num_subcores=NS)
_sc_params = pltpu.CompilerParams(use_tc_tiling_on_sc=False,
                                  needs_layout_passes=False)


@functools.partial(
    pl.kernel,
    out_type=jax.ShapeDtypeStruct((NC, NRP, 16), jnp.float32),
    mesh=_mesh,
    compiler_params=_sc_params,
    scratch_types=[
        pltpu.VMEM((EPT,), jnp.int32),
        pltpu.VMEM((KCH, CH), jnp.int32),
        pltpu.VMEM((CH, 16), jnp.float32),
        pltpu.VMEM_SHARED((NRP, 16), jnp.float32),
        pltpu.SemaphoreType.DMA,
    ],
)
def _edge_sums1(xpad_hbm, src_hbm, seg_hbm, zeros_hbm, out_hbm,
                srcbuf, segbuf, rows, acc, sem):
    c = lax.axis_index("c")
    s = lax.axis_index("s")
    wid = c * NS + s
    base = wid * EPT

    pltpu.sync_copy(src_hbm.at[pl.ds(base, EPT)], srcbuf)
    pltpu.sync_copy(seg_hbm.at[pl.ds(wid * KCH, KCH)], segbuf)

    pltpu.sync_copy(zeros_hbm.at[pl.ds(s * ROWS_A, ROWS_A)],
                    acc.at[pl.ds(s * ROWS_A, ROWS_A)])
    plsc.subcore_barrier()

    @pl.loop(0, KCH)
    def _chunk(j):
        pltpu.async_copy(xpad_hbm.at[srcbuf.at[pl.ds(j * CH, CH)]],
                         rows, sem).wait()
        pltpu.sync_copy(rows, acc.at[segbuf.at[j]], add=True)

    plsc.subcore_barrier()
    pltpu.sync_copy(acc.at[pl.ds(s * ROWS_A, ROWS_A)],
                    out_hbm.at[c, pl.ds(s * ROWS_A, ROWS_A)])


@functools.partial(
    pl.kernel,
    out_type=jax.ShapeDtypeStruct((EPAD,), jnp.float32),
    mesh=_mesh,
    compiler_params=_sc_params,
    scratch_types=[
        pltpu.VMEM((NR + 16,), jnp.float32),
        pltpu.VMEM((EPT,), jnp.int32),
        pltpu.VMEM((EPT,), jnp.float32),
    ],
)
def _edge_alpha(ic_hbm, seg_hbm, out_hbm, ictab, segbuf, abuf):
    c = lax.axis_index("c")
    s = lax.axis_index("s")
    wid = c * NS + s
    base = wid * EPT
    pltpu.sync_copy(ic_hbm, ictab)
    pltpu.sync_copy(seg_hbm.at[pl.ds(base, EPT)], segbuf)

    @pl.loop(0, EPT // 16)
    def _grp(j):
        segv = segbuf[pl.ds(j * 16, 16)]
        abuf[pl.ds(j * 16, 16)] = plsc.load_gather(ictab, [segv])

    pltpu.sync_copy(abuf, out_hbm.at[pl.ds(base, EPT)])


@functools.partial(
    pl.kernel,
    out_type=jax.ShapeDtypeStruct((NC, NP, D), jnp.float32),
    mesh=_mesh,
    compiler_params=_sc_params,
    scratch_types=[
        pltpu.VMEM((EPT,), jnp.int32),
        pltpu.VMEM((KCH, CH), jnp.int32),
        pltpu.VMEM((EPT,), jnp.float32),
        pltpu.VMEM((CH, D), jnp.float32),
        pltpu.VMEM_SHARED((NP, D), jnp.float32),
        pltpu.SemaphoreType.DMA,
    ],
)
def _edge_sums2(t2_hbm, tix_hbm, dst_hbm, alpha_hbm, zeros_hbm, out_hbm,
                tixbuf, dstbuf, abuf, rows, acc, sem):
    c = lax.axis_index("c")
    s = lax.axis_index("s")
    wid = c * NS + s
    base = wid * EPT

    pltpu.sync_copy(tix_hbm.at[pl.ds(base, EPT)], tixbuf)
    pltpu.sync_copy(dst_hbm.at[pl.ds(wid * KCH, KCH)], dstbuf)
    pltpu.sync_copy(alpha_hbm.at[pl.ds(base, EPT)], abuf)

    pltpu.sync_copy(zeros_hbm.at[pl.ds(s * ROWS_C, ROWS_C)],
                    acc.at[pl.ds(s * ROWS_C, ROWS_C)])
    plsc.subcore_barrier()

    @pl.loop(0, KCH)
    def _chunk(j):
        pltpu.async_copy(t2_hbm.at[tixbuf.at[pl.ds(j * CH, CH)]],
                         rows, sem).wait()

        @pl.loop(0, CH // 16)
        def _grp(g):
            av = abuf[pl.ds(j * CH + g * 16, 16)]
            for l in range(16):
                e = g * 16 + l
                a = av[l]

                @pl.loop(0, D // 16)
                def _seg(k):
                    rows[e, pl.ds(k * 16, 16)] = rows[e, pl.ds(k * 16, 16)] * a

        pltpu.sync_copy(rows, acc.at[dstbuf.at[j]], add=True)

    plsc.subcore_barrier()
    pltpu.sync_copy(acc.at[pl.ds(s * ROWS_C, ROWS_C)],
                    out_hbm.at[c, pl.ds(s * ROWS_C, ROWS_C)])


def kernel(x, edge_index, edge_type, batch, W1, root1, b1, W2, root2, b2,
           gate_W, gate_b, fc_W, fc_b):
    src = edge_index[0]
    dst = edge_index[1]
    seg = dst * R + edge_type
    npad = EPAD - E

    xpad = jnp.zeros((N + 1, 16), jnp.float32)
    xpad = xpad.at[:N, :DIN].set(x).at[:N, DIN].set(1.0)

    src_p = jnp.concatenate([src, jnp.full((npad,), N, jnp.int32)])
    seg_p = jnp.concatenate([seg, jnp.zeros((npad,), jnp.int32)])
    zeros_a = jnp.zeros((NRP, 16), jnp.float32)

    s2 = _edge_sums1(xpad, src_p, seg_p.reshape(NW * KCH, CH), zeros_a)
    S = s2[0, :NR] + s2[1, :NR]
    cnt = S[:, DIN]
    ic = 1.0 / jnp.maximum(cnt, 1.0)
    mean = S[:, :DIN] * ic[:, None]

    agg1 = mean.reshape(N, R * DIN) @ W1.reshape(R * DIN, D)
    h1 = jax.nn.relu(agg1 + x @ root1 + b1)

    t2 = (h1 @ jnp.transpose(W2, (1, 0, 2)).reshape(D, R * D)).reshape(NR, D)

    ic_pad = jnp.concatenate([ic, jnp.zeros((16,), jnp.float32)])
    aseg_p = jnp.concatenate([seg, jnp.full((npad,), NR, jnp.int32)])
    alpha = _edge_alpha(ic_pad, aseg_p)

    tix_p = jnp.concatenate([src * R + edge_type, jnp.zeros((npad,), jnp.int32)])
    dst_p = jnp.concatenate([dst, jnp.zeros((npad,), jnp.int32)])
    zeros_c = jnp.zeros((NP, D), jnp.float32)

    a2 = _edge_sums2(t2, tix_p, dst_p.reshape(NW * KCH, CH), alpha, zeros_c)
    agg2 = a2[0, :N] + a2[1, :N]
    h2 = jax.nn.relu(agg2 + h1 @ root2 + b2)

    gate = (h2 @ gate_W + gate_b)[:, 0]
    onehot = (batch[None, :] == jnp.arange(G, dtype=jnp.int32)[:, None])
    gmax = jnp.max(jnp.where(onehot, gate[None, :], -jnp.inf), axis=1)
    ge = jnp.exp(gate - gmax[batch])
    denom = onehot.astype(jnp.float32) @ ge
    w = ge / denom[batch]
    pooled = onehot.astype(jnp.float32) @ (w[:, None] * h2)
    return pooled @ fc_W + fc_b
